# baseline (speedup 1.0000x reference)
import functools

import jax
import jax.numpy as jnp
from jax import lax
from jax.experimental import pallas as pl
from jax.experimental.pallas import tpu as pltpu
from jax.experimental.pallas import tpu_sc as plsc

N_VOCAB = 49408
N_EMBD = 768
N_TOKENS = 77
BATCH = 256

NC = 2
NS = 16
L = 16
NW = NC * NS
CB = 32
QPT = BATCH // CB
NCHUNK = N_TOKENS * QPT
KMAX = 20
KPAD = 24
NSTEP = 21
NVEC = N_EMBD // L
JB = 16


def _make_kernel():
  mesh = plsc.VectorSubcoreMesh(core_axis_name="c", subcore_axis_name="s")

  @functools.partial(
      pl.kernel,
      mesh=mesh,
      out_type=jax.ShapeDtypeStruct((N_TOKENS, BATCH, N_EMBD), jnp.float32),
      scratch_types=[
          pltpu.VMEM((KMAX, CB), jnp.int32),
          pltpu.VMEM((1, KPAD), jnp.int32),
          pltpu.VMEM((KPAD, N_EMBD), jnp.float32),
          pltpu.VMEM((CB, N_EMBD), jnp.float32),
          pltpu.VMEM((CB, N_EMBD), jnp.float32),
          pltpu.VMEM((CB, N_EMBD), jnp.float32),
          pltpu.SemaphoreType.DMA,
          pltpu.SemaphoreType.DMA,
          pltpu.SemaphoreType.DMA,
          pltpu.SemaphoreType.DMA,
          pltpu.SemaphoreType.DMA,
          pltpu.SemaphoreType.DMA,
          pltpu.SemaphoreType.DMA,
      ],
  )
  def emb_kernel(tidx_hbm, idx_hbm, table_hbm, pos_hbm, out_hbm, idx_v,
                 ptidx_v, pos_v, buf0, buf1, buf2, gsem0, gsem1, gsem2,
                 wsem0, wsem1, wsem2, psem):
    bufs = (buf0, buf1, buf2)
    gsems = (gsem0, gsem1, gsem2)
    wsems = (wsem0, wsem1, wsem2)
    wid = lax.axis_index("s") * NC + lax.axis_index("c")

    def gather_of(k, b):
      return pltpu.make_async_copy(
          table_hbm.at[idx_v.at[k]], bufs[b], gsems[b])

    def write_of(k, b):
      g = wid + NW * k
      t = g // QPT
      qoff = pl.multiple_of((g % QPT) * CB, CB)
      return pltpu.make_async_copy(
          bufs[b], out_hbm.at[t, pl.ds(qoff, CB)], wsems[b])

    def valid(k):
      return wid + NW * k < NCHUNK

    pltpu.sync_copy(idx_hbm.at[wid], idx_v)
    gather_of(0, 0).start()
    gather_of(1, 1).start()
    pltpu.sync_copy(tidx_hbm.at[wid], ptidx_v)
    pltpu.async_copy(pos_hbm.at[ptidx_v.at[0]], pos_v, psem).wait()

    def per_round(r, _):
      for s in range(3):
        k = r * 3 + s

        @pl.when(valid(k))
        def _():
          gather_of(k, s).wait()

        pred_w = valid(k - 1) if s > 0 else jnp.logical_and(r >= 1,
                                                            valid(k - 1))
        @pl.when(pred_w)
        def _():
          write_of(k - 1, (s + 2) % 3).wait()

        @pl.when(valid(k + 2))
        def _():
          gather_of(k + 2, (s + 2) % 3).start()

        @pl.when(valid(k))
        def _():
          for jb in range(NVEC // JB):
            pregs = [pos_v[k, pl.ds((jb * JB + j) * L, L)] for j in range(JB)]

            def add_block(i, c):
              for j in range(JB):
                plsc.addupdate(bufs[s].at[i, pl.ds((jb * JB + j) * L, L)],
                               pregs[j])
              return c

            lax.fori_loop(0, CB, add_block, 0)
          write_of(k, s).start()
      return 0

    lax.fori_loop(0, NSTEP // 3, per_round, 0)

  return emb_kernel


_EMB_KERNEL = _make_kernel()


def kernel(tokens, token_embedding, position_embedding):
  tok_t = tokens.astype(jnp.int32).T
  idx_all = tok_t.reshape(NCHUNK, CB)
  idx_all = jnp.pad(idx_all, ((0, KMAX * NW - NCHUNK), (0, 0)))
  idx_all = idx_all.reshape(KMAX, NW, CB).transpose(1, 0, 2)
  g = jnp.arange(NW, dtype=jnp.int32)[:, None] + NW * jnp.arange(
      KPAD, dtype=jnp.int32)[None, :]
  tidx_all = jnp.minimum(g // QPT, N_TOKENS - 1).reshape(NW, 1, KPAD)
  out_t = _EMB_KERNEL(tidx_all, idx_all, token_embedding,
                      position_embedding)
  return jnp.transpose(out_t, (1, 0, 2))


# speedup vs baseline: 1.1210x; 1.1210x over previous
import functools

import jax
import jax.numpy as jnp
from jax import lax
from jax.experimental import pallas as pl
from jax.experimental.pallas import tpu as pltpu
from jax.experimental.pallas import tpu_sc as plsc

N_VOCAB = 49408
N_EMBD = 768
N_TOKENS = 77
BATCH = 256

NC = 2
NS = 16
L = 16
NW = NC * NS
CB = 32
QPT = BATCH // CB
NCHUNK = N_TOKENS * QPT
KMAX = 20
NSTEP = 21
NVEC = N_EMBD // L
JB = 16


def _make_kernel():
  mesh = plsc.VectorSubcoreMesh(core_axis_name="c", subcore_axis_name="s")

  @functools.partial(
      pl.kernel,
      mesh=mesh,
      out_type=jax.ShapeDtypeStruct((N_TOKENS, BATCH, N_EMBD), jnp.float32),
      scratch_types=[
          pltpu.VMEM((KMAX, CB), jnp.int32),
          pltpu.VMEM((KMAX, N_EMBD), jnp.float32),
          pltpu.VMEM((CB, N_EMBD), jnp.float32),
          pltpu.VMEM((CB, N_EMBD), jnp.float32),
          pltpu.VMEM((CB, N_EMBD), jnp.float32),
          pltpu.SemaphoreType.DMA,
          pltpu.SemaphoreType.DMA,
          pltpu.SemaphoreType.DMA,
          pltpu.SemaphoreType.DMA,
          pltpu.SemaphoreType.DMA,
          pltpu.SemaphoreType.DMA,
      ],
  )
  def emb_kernel(idx_hbm, table_hbm, pos_hbm, out_hbm, idx_v, pos_v,
                 buf0, buf1, buf2, gsem0, gsem1, gsem2, wsem0, wsem1, wsem2):
    bufs = (buf0, buf1, buf2)
    gsems = (gsem0, gsem1, gsem2)
    wsems = (wsem0, wsem1, wsem2)
    wid = lax.axis_index("s") * NC + lax.axis_index("c")

    def gather_of(k, b):
      return pltpu.make_async_copy(
          table_hbm.at[idx_v.at[k]], bufs[b], gsems[b])

    def write_of(k, b):
      g = wid + NW * k
      t = g // QPT
      qoff = pl.multiple_of((g % QPT) * CB, CB)
      return pltpu.make_async_copy(
          bufs[b], out_hbm.at[t, pl.ds(qoff, CB)], wsems[b])

    def valid(k):
      return wid + NW * k < NCHUNK

    pltpu.sync_copy(idx_hbm.at[wid], idx_v)
    gather_of(0, 0).start()
    gather_of(1, 1).start()
    pltpu.sync_copy(pos_hbm.at[wid], pos_v)

    def per_round(r, _):
      for s in range(3):
        k = r * 3 + s

        @pl.when(valid(k))
        def _():
          gather_of(k, s).wait()

        pred_w = valid(k - 1) if s > 0 else jnp.logical_and(r >= 1,
                                                            valid(k - 1))
        @pl.when(pred_w)
        def _():
          write_of(k - 1, (s + 2) % 3).wait()

        @pl.when(valid(k + 2))
        def _():
          gather_of(k + 2, (s + 2) % 3).start()

        @pl.when(valid(k))
        def _():
          for jb in range(NVEC // JB):
            pregs = [pos_v[k, pl.ds((jb * JB + j) * L, L)] for j in range(JB)]

            def add_block(i, c):
              for j in range(JB):
                plsc.addupdate(bufs[s].at[i, pl.ds((jb * JB + j) * L, L)],
                               pregs[j])
              return c

            lax.fori_loop(0, CB, add_block, 0)
          write_of(k, s).start()
      return 0

    lax.fori_loop(0, NSTEP // 3, per_round, 0)

  return emb_kernel


_EMB_KERNEL = _make_kernel()


def kernel(tokens, token_embedding, position_embedding):
  tok_t = tokens.astype(jnp.int32).T
  idx_all = tok_t.reshape(NCHUNK, CB)
  idx_all = jnp.pad(idx_all, ((0, KMAX * NW - NCHUNK), (0, 0)))
  idx_all = idx_all.reshape(KMAX, NW, CB).transpose(1, 0, 2)
  pos_all = jnp.repeat(position_embedding, QPT, axis=0)
  pos_all = jnp.pad(pos_all, ((0, KMAX * NW - NCHUNK), (0, 0)))
  pos_all = pos_all.reshape(KMAX, NW, N_EMBD).transpose(1, 0, 2)
  out_t = _EMB_KERNEL(idx_all, token_embedding, pos_all)
  return jnp.transpose(out_t, (1, 0, 2))
